# baseline (speedup 1.0000x reference)
import jax
import jax.numpy as jnp
from jax.experimental import pallas as pl
from jax.experimental.pallas import tpu as pltpu


def kernel(inputs, pos_table):
    N, T = inputs.shape
    H = pos_table.shape[1]
    scale = float(H) ** 0.5
    flat = pos_table.reshape(1, T * H)

    BN = 128
    NB = N // BN

    def body(tab_ref, out_ref, buf, sems):
        buf[...] = jnp.broadcast_to(tab_ref[...] * scale, buf.shape)
        for i in range(NB):
            pltpu.make_async_copy(
                buf, out_ref.at[pl.ds(i * BN, BN), :], sems.at[i]
            ).start()
        for i in range(NB):
            pltpu.make_async_copy(
                buf, out_ref.at[pl.ds(i * BN, BN), :], sems.at[i]
            ).wait()

    out = pl.pallas_call(
        body,
        in_specs=[pl.BlockSpec(memory_space=pltpu.MemorySpace.VMEM)],
        out_specs=pl.BlockSpec(memory_space=pl.ANY),
        out_shape=jax.ShapeDtypeStruct((N, T * H), jnp.float32),
        scratch_shapes=[
            pltpu.VMEM((BN, T * H), jnp.float32),
            pltpu.SemaphoreType.DMA((NB,)),
        ],
    )(flat)
    return out.reshape(N, T, H)
